# Initial kernel scaffold; baseline (speedup 1.0000x reference)
import functools

import jax
import jax.numpy as jnp
from jax import lax
from jax.experimental import pallas as pl
from jax.experimental.pallas import tpu as pltpu
from jax.experimental.pallas import tpu_sc as plsc

N = 10000
E = 320000
D = 128
H = 128
HEADS = 8
DH = 16
NPAT = 6

NC = 2
NS = 16
NW = NC * NS
LK = 128
ETOT = E + N
NB_W = 81
EW = NB_W * LK
EPAD = NW * EW
N_PAD = 10016
ROWS_T = N_PAD // NS
EPS_GN = 1e-5


def _mesh():
    return plsc.VectorSubcoreMesh(
        core_axis_name="c", subcore_axis_name="s", num_cores=NC, num_subcores=NS)


def _sc_degree(cols2, ones16, zeros16):
    @functools.partial(
        pl.kernel,
        out_type=jax.ShapeDtypeStruct((NC * N_PAD, 16), jnp.float32),
        mesh=_mesh(),
        scratch_types=[
            pltpu.VMEM_SHARED((N_PAD, 16), jnp.float32),
            pltpu.VMEM((NB_W, LK), jnp.int32),
            pltpu.VMEM((LK, 16), jnp.float32),
        ],
    )
    def k(cols_hbm, ones_hbm, zeros_hbm, out_hbm, acc, cols_v, ones_v):
        c = lax.axis_index("c")
        s = lax.axis_index("s")
        w = c * NS + s
        pltpu.sync_copy(cols_hbm.at[pl.ds(w * NB_W, NB_W)], cols_v)
        pltpu.sync_copy(ones_hbm, ones_v)
        pltpu.sync_copy(zeros_hbm, acc.at[pl.ds(s * ROWS_T, ROWS_T)])
        plsc.subcore_barrier()

        def body(nb, carry):
            pltpu.sync_copy(ones_v, acc.at[cols_v.at[nb]], add=True)
            return carry

        lax.fori_loop(0, NB_W, body, 0)
        plsc.subcore_barrier()
        pltpu.sync_copy(acc.at[pl.ds(s * ROWS_T, ROWS_T)],
                        out_hbm.at[pl.ds(c * N_PAD + s * ROWS_T, ROWS_T)])

    return k(cols2, ones16, zeros16)


def _sc_gcn(hd, rows2, cols2, zeros128):
    @functools.partial(
        pl.kernel,
        out_type=jax.ShapeDtypeStruct((NC * N_PAD, H), jnp.float32),
        mesh=_mesh(),
        scratch_types=[
            pltpu.VMEM_SHARED((N_PAD, H), jnp.float32),
            pltpu.VMEM((NB_W, LK), jnp.int32),
            pltpu.VMEM((NB_W, LK), jnp.int32),
            pltpu.VMEM((LK, H), jnp.float32),
            pltpu.SemaphoreType.DMA,
        ],
    )
    def k(hd_hbm, rows_hbm, cols_hbm, zeros_hbm, out_hbm,
          acc, rows_v, cols_v, buf, sem):
        c = lax.axis_index("c")
        s = lax.axis_index("s")
        w = c * NS + s
        pltpu.sync_copy(rows_hbm.at[pl.ds(w * NB_W, NB_W)], rows_v)
        pltpu.sync_copy(cols_hbm.at[pl.ds(w * NB_W, NB_W)], cols_v)
        pltpu.sync_copy(zeros_hbm, acc.at[pl.ds(s * ROWS_T, ROWS_T)])
        plsc.subcore_barrier()

        def body(nb, carry):
            pltpu.async_copy(hd_hbm.at[rows_v.at[nb]], buf, sem).wait()
            pltpu.sync_copy(buf, acc.at[cols_v.at[nb]], add=True)
            return carry

        lax.fori_loop(0, NB_W, body, 0)
        plsc.subcore_barrier()
        pltpu.sync_copy(acc.at[pl.ds(s * ROWS_T, ROWS_T)],
                        out_hbm.at[pl.ds(c * N_PAD + s * ROWS_T, ROWS_T)])

    return k(hd, rows2, cols2, zeros128)


def _sc_gat(hm, as16, ad16, m16, rows2, cols2, zeros128, zeros16):
    @functools.partial(
        pl.kernel,
        out_type=(jax.ShapeDtypeStruct((NC * N_PAD, H), jnp.float32),
                  jax.ShapeDtypeStruct((NC * N_PAD, 16), jnp.float32)),
        mesh=_mesh(),
        scratch_types=[
            pltpu.VMEM_SHARED((N_PAD, H), jnp.float32),
            pltpu.VMEM_SHARED((N_PAD, 16), jnp.float32),
            pltpu.VMEM((NB_W, LK), jnp.int32),
            pltpu.VMEM((NB_W, LK), jnp.int32),
            pltpu.VMEM((LK, H), jnp.float32),
            pltpu.VMEM((LK, 16), jnp.float32),
            pltpu.VMEM((LK, 16), jnp.float32),
            pltpu.VMEM((LK, 16), jnp.float32),
            pltpu.VMEM((16,), jnp.float32),
            pltpu.SemaphoreType.DMA,
            pltpu.SemaphoreType.DMA,
            pltpu.SemaphoreType.DMA,
        ],
    )
    def k(hm_hbm, as_hbm, ad_hbm, m16_hbm, rows_hbm, cols_hbm,
          z128_hbm, z16_hbm, out_hbm, den_hbm,
          acc, den, rows_v, cols_v, hbuf, asb, adb, exb, m16v,
          sem_h, sem_a, sem_b):
        c = lax.axis_index("c")
        s = lax.axis_index("s")
        w = c * NS + s
        pltpu.sync_copy(rows_hbm.at[pl.ds(w * NB_W, NB_W)], rows_v)
        pltpu.sync_copy(cols_hbm.at[pl.ds(w * NB_W, NB_W)], cols_v)
        pltpu.sync_copy(m16_hbm, m16v)
        pltpu.sync_copy(z128_hbm, acc.at[pl.ds(s * ROWS_T, ROWS_T)])
        pltpu.sync_copy(z16_hbm, den.at[pl.ds(s * ROWS_T, ROWS_T)])
        plsc.subcore_barrier()

        def body(nb, carry):
            cp_h = pltpu.async_copy(hm_hbm.at[rows_v.at[nb]], hbuf, sem_h)
            cp_a = pltpu.async_copy(as_hbm.at[rows_v.at[nb]], asb, sem_a)
            cp_b = pltpu.async_copy(ad_hbm.at[cols_v.at[nb]], adb, sem_b)
            cp_a.wait()
            cp_b.wait()
            m = m16v[...]

            def edge(j, carry2):
                v = asb[j] + adb[j]
                v = jnp.maximum(v, v * 0.2)
                exb[j] = jnp.exp(v - m)
                return carry2

            lax.fori_loop(0, LK, edge, 0)
            cp_h.wait()

            def scale(j, carry2):
                for hd in range(HEADS):
                    sv = exb[j, hd]
                    blk = hbuf[j, pl.ds(hd * DH, DH)]
                    hbuf[j, pl.ds(hd * DH, DH)] = blk * sv
                return carry2

            lax.fori_loop(0, LK, scale, 0)
            pltpu.sync_copy(exb, den.at[cols_v.at[nb]], add=True)
            pltpu.sync_copy(hbuf, acc.at[cols_v.at[nb]], add=True)
            return carry

        lax.fori_loop(0, NB_W, body, 0)
        plsc.subcore_barrier()
        pltpu.sync_copy(acc.at[pl.ds(s * ROWS_T, ROWS_T)],
                        out_hbm.at[pl.ds(c * N_PAD + s * ROWS_T, ROWS_T)])
        pltpu.sync_copy(den.at[pl.ds(s * ROWS_T, ROWS_T)],
                        den_hbm.at[pl.ds(c * N_PAD + s * ROWS_T, ROWS_T)])

    return k(hm, as16, ad16, m16, rows2, cols2, zeros128, zeros16)


def _graphnorm(h, p_w, p_b, p_ms):
    mean = jnp.mean(h, axis=0, keepdims=True)
    xc = h - mean * p_ms
    var = jnp.mean(xc * xc, axis=0, keepdims=True)
    return p_w * xc * lax.rsqrt(var + EPS_GN) + p_b


def _tc1(x, dega, degb, we, be, w0):
    def body(x_r, dega_r, degb_r, we_r, be_r, w0_r,
             xemb_o, hd0_o, dinv_o):
        deg = dega_r[:, 0:1] + degb_r[:, 0:1]
        dinv = jnp.where(deg > 0, lax.rsqrt(deg), 0.0)
        xemb = jnp.dot(x_r[...], we_r[...],
                       preferred_element_type=jnp.float32) + be_r[...]
        hd0 = dinv * jnp.dot(xemb, w0_r[...],
                             preferred_element_type=jnp.float32)
        xemb_o[...] = xemb
        hd0_o[...] = hd0
        dinv_o[...] = dinv

    return pl.pallas_call(
        body,
        out_shape=(jax.ShapeDtypeStruct((N, H), jnp.float32),
                   jax.ShapeDtypeStruct((N, H), jnp.float32),
                   jax.ShapeDtypeStruct((N, 1), jnp.float32)),
    )(x, dega, degb, we, be, w0)


def _tc2(p0a, p0b, dinv, b0, n0w, n0b, n0ms, xemb, w1, att_s, att_d, msel):
    def body(p0a_r, p0b_r, dinv_r, b0_r, n0w_r, n0b_r, n0ms_r, xemb_r,
             w1_r, atts_r, attd_r, msel_r,
             x1_o, hm_o, as_o, ad_o, m16_o):
        h = dinv_r[...] * (p0a_r[...] + p0b_r[...]) + b0_r[...]
        h = _graphnorm(h, n0w_r[...], n0b_r[...], n0ms_r[...])
        x1 = jnp.maximum(h, 0.0) + xemb_r[...]
        hm = jnp.dot(x1, w1_r[...], preferred_element_type=jnp.float32)
        a_s = jnp.dot(hm * atts_r[...], msel_r[...],
                      preferred_element_type=jnp.float32)
        a_d = jnp.dot(hm * attd_r[...], msel_r[...],
                      preferred_element_type=jnp.float32)
        mm = (jnp.max(a_s, axis=0, keepdims=True)
              + jnp.max(a_d, axis=0, keepdims=True))
        mub = jnp.maximum(mm, mm * 0.2)
        zero8 = jnp.zeros((N, 8), jnp.float32)
        x1_o[...] = x1
        hm_o[...] = hm
        as_o[...] = jnp.concatenate([a_s, zero8], axis=1)
        ad_o[...] = jnp.concatenate([a_d, zero8], axis=1)
        m16_o[...] = jnp.concatenate([mub, jnp.zeros((1, 8), jnp.float32)],
                                     axis=1)

    return pl.pallas_call(
        body,
        out_shape=(jax.ShapeDtypeStruct((N, H), jnp.float32),
                   jax.ShapeDtypeStruct((N, H), jnp.float32),
                   jax.ShapeDtypeStruct((N, 16), jnp.float32),
                   jax.ShapeDtypeStruct((N, 16), jnp.float32),
                   jax.ShapeDtypeStruct((1, 16), jnp.float32)),
    )(p0a, p0b, dinv, b0, n0w, n0b, n0ms, xemb, w1, att_s, att_d, msel)


def _tc4(paa, pab, dena, denb, b1, n1w, n1b, n1ms, x1, w2, dinv, mexp):
    def body(paa_r, pab_r, dena_r, denb_r, b1_r, n1w_r, n1b_r, n1ms_r,
             x1_r, w2_r, dinv_r, mexp_r, x2_o, hd2_o):
        den = dena_r[:, 0:8] + denb_r[:, 0:8]
        scale = 1.0 / (den + 1e-16)
        scale128 = jnp.dot(scale, mexp_r[...],
                           preferred_element_type=jnp.float32)
        h = (paa_r[...] + pab_r[...]) * scale128 + b1_r[...]
        h = _graphnorm(h, n1w_r[...], n1b_r[...], n1ms_r[...])
        x2 = jnp.maximum(h, 0.0) + x1_r[...]
        hd2 = dinv_r[...] * jnp.dot(x2, w2_r[...],
                                    preferred_element_type=jnp.float32)
        x2_o[...] = x2
        hd2_o[...] = hd2

    return pl.pallas_call(
        body,
        out_shape=(jax.ShapeDtypeStruct((N, H), jnp.float32),
                   jax.ShapeDtypeStruct((N, H), jnp.float32)),
    )(paa, pab, dena, denb, b1, n1w, n1b, n1ms, x1, w2, dinv, mexp)


def _tc5(p2a, p2b, dinv, b2, n2w, n2b, n2ms, x2, x, wlist):
    def body(p2a_r, p2b_r, dinv_r, b2_r, n2w_r, n2b_r, n2ms_r, x2_r, x_r,
             wi1x_r, wi1s_r, bi1_r, wi2_r, bi2_r,
             ws1x_r, ws1h_r, bs1_r, ws2_r, bs2_r,
             wp1x_r, wp1gm_r, wp1gx_r, wp1s_r, bp1_r,
             wp2_r, bp2_r, wp3_r, bp3_r,
             wt1m_r, wt1x_r, bt1_r, wt2_r, bt2_r,
             hl_o, hp_o, plg_o, pp_o, tl_o, tp_o, hi_o, xe_o):
        h = dinv_r[...] * (p2a_r[...] + p2b_r[...]) + b2_r[...]
        h = _graphnorm(h, n2w_r[...], n2b_r[...], n2ms_r[...])
        xe = jnp.maximum(h, 0.0) + x2_r[...]
        structural = x_r[:, 0:6]

        hi1 = jnp.dot(xe, wi1x_r[...], preferred_element_type=jnp.float32)
        hi1 = hi1 + jnp.dot(structural, wi1s_r[...],
                            preferred_element_type=jnp.float32) + bi1_r[...]
        hi1 = jnp.maximum(hi1, 0.0)
        himp_l = jnp.dot(hi1, wi2_r[...],
                         preferred_element_type=jnp.float32) + bi2_r[...]
        himp = 1.0 / (1.0 + jnp.exp(-himp_l))

        hs1 = jnp.dot(xe, ws1x_r[...], preferred_element_type=jnp.float32)
        hs1 = hs1 + himp * ws1h_r[...] + bs1_r[...]
        hs1 = jnp.maximum(hs1, 0.0)
        hub_l = (jnp.dot(hs1, ws2_r[...], preferred_element_type=jnp.float32)
                 + bs2_r[...] + 2.0 * himp)
        mx = jnp.max(hub_l, axis=0, keepdims=True)
        ex = jnp.exp(hub_l - mx)
        hub_p = ex / jnp.sum(ex, axis=0, keepdims=True)

        gmean = jnp.mean(xe, axis=0, keepdims=True)
        gmax = jnp.max(xe, axis=0, keepdims=True)

        pat = jnp.dot(xe, wp1x_r[...], preferred_element_type=jnp.float32)
        gfp = (jnp.dot(gmean, wp1gm_r[...], preferred_element_type=jnp.float32)
               + jnp.dot(gmax, wp1gx_r[...], preferred_element_type=jnp.float32))
        pat = pat + gfp + jnp.dot(structural, wp1s_r[...],
                                  preferred_element_type=jnp.float32) + bp1_r[...]
        pat = jnp.maximum(pat, 0.0)
        pat = jnp.maximum(jnp.dot(pat, wp2_r[...],
                                  preferred_element_type=jnp.float32)
                          + bp2_r[...], 0.0)
        pat_l = jnp.dot(pat, wp3_r[...],
                        preferred_element_type=jnp.float32) + bp3_r[...]
        pm = jnp.max(pat_l, axis=1, keepdims=True)
        pe = jnp.exp(pat_l - pm)
        pat_p = pe / jnp.sum(pe, axis=1, keepdims=True)

        t = (jnp.dot(gmean, wt1m_r[...], preferred_element_type=jnp.float32)
             + jnp.dot(gmax, wt1x_r[...], preferred_element_type=jnp.float32)
             + bt1_r[...])
        t = jnp.maximum(t, 0.0)
        term_l = jnp.dot(t, wt2_r[...],
                         preferred_element_type=jnp.float32) + bt2_r[...]
        tm = jnp.max(term_l, axis=1, keepdims=True)
        te = jnp.exp(term_l - tm)
        term_p = te / jnp.sum(te, axis=1, keepdims=True)

        hl_o[...] = hub_l
        hp_o[...] = hub_p
        plg_o[...] = pat_l
        pp_o[...] = pat_p
        tl_o[...] = term_l
        tp_o[...] = term_p
        hi_o[...] = himp
        xe_o[...] = xe

    return pl.pallas_call(
        body,
        out_shape=(jax.ShapeDtypeStruct((N, 1), jnp.float32),
                   jax.ShapeDtypeStruct((N, 1), jnp.float32),
                   jax.ShapeDtypeStruct((N, NPAT), jnp.float32),
                   jax.ShapeDtypeStruct((N, NPAT), jnp.float32),
                   jax.ShapeDtypeStruct((1, 2), jnp.float32),
                   jax.ShapeDtypeStruct((1, 2), jnp.float32),
                   jax.ShapeDtypeStruct((N, 1), jnp.float32),
                   jax.ShapeDtypeStruct((N, H), jnp.float32)),
    )(p2a, p2b, dinv, b2, n2w, n2b, n2ms, x2, x, *wlist)


def kernel(x, edge_index, batch, params):
    del batch
    f32 = jnp.float32

    loops = jnp.arange(N, dtype=jnp.int32)
    pad = EPAD - ETOT
    rows = jnp.concatenate([edge_index[0], loops,
                            jnp.zeros((pad,), jnp.int32)])
    cols = jnp.concatenate([edge_index[1], loops,
                            jnp.full((pad,), N, jnp.int32)])
    rows2 = rows.reshape(EPAD // LK, LK)
    cols2 = cols.reshape(EPAD // LK, LK)

    zeros128 = jnp.zeros((ROWS_T, H), f32)
    zeros16 = jnp.zeros((ROWS_T, 16), f32)
    ones16 = jnp.ones((LK, 16), f32)

    msel = jnp.repeat(jnp.eye(HEADS, dtype=f32), DH, axis=0)
    mexp = msel.T

    p = params
    row = lambda v: v.reshape(1, -1)

    degout = _sc_degree(cols2, ones16, zeros16)
    dega = degout[0:N]
    degb = degout[N_PAD:N_PAD + N]

    xemb, hd0, dinv = _tc1(x, dega, degb,
                           p["node_embed"]["w"], row(p["node_embed"]["b"]),
                           p["gcn0"]["w"])

    p0 = _sc_gcn(hd0, rows2, cols2, zeros128)
    p0a, p0b = p0[0:N], p0[N_PAD:N_PAD + N]

    x1, hm, as16, ad16, m16 = _tc2(
        p0a, p0b, dinv, row(p["gcn0"]["b"]),
        row(p["norm0"]["weight"]), row(p["norm0"]["bias"]),
        row(p["norm0"]["mean_scale"]), xemb,
        p["gat1"]["w"], row(p["gat1"]["att_src"].reshape(-1)),
        row(p["gat1"]["att_dst"].reshape(-1)), msel)

    pa, denp = _sc_gat(hm, as16, ad16, m16.reshape(16), rows2, cols2,
                       zeros128, zeros16)
    paa, pab = pa[0:N], pa[N_PAD:N_PAD + N]
    dena, denb = denp[0:N], denp[N_PAD:N_PAD + N]

    x2, hd2 = _tc4(paa, pab, dena, denb, row(p["gat1"]["b"]),
                   row(p["norm1"]["weight"]), row(p["norm1"]["bias"]),
                   row(p["norm1"]["mean_scale"]), x1, p["gcn2"]["w"],
                   dinv, mexp)

    p2 = _sc_gcn(hd2, rows2, cols2, zeros128)
    p2a, p2b = p2[0:N], p2[N_PAD:N_PAD + N]

    wi1 = p["hub_imp1"]["w"]
    ws1 = p["hub_sel1"]["w"]
    pw1 = p["pat1"]["w"]
    tw1 = p["term1"]["w"]
    wlist = (
        wi1[0:H], wi1[H:H + 6], row(p["hub_imp1"]["b"]),
        p["hub_imp2"]["w"], row(p["hub_imp2"]["b"]),
        ws1[0:H], row(ws1[H]), row(p["hub_sel1"]["b"]),
        p["hub_sel2"]["w"], row(p["hub_sel2"]["b"]),
        pw1[0:H], pw1[H:2 * H], pw1[2 * H:3 * H], pw1[3 * H:3 * H + 6],
        row(p["pat1"]["b"]),
        p["pat2"]["w"], row(p["pat2"]["b"]),
        p["pat3"]["w"], row(p["pat3"]["b"]),
        tw1[0:H], tw1[H:2 * H], row(p["term1"]["b"]),
        p["term2"]["w"], row(p["term2"]["b"]),
    )
    (hub_l, hub_p, pat_l, pat_p, term_l, term_p, himp, xe) = _tc5(
        p2a, p2b, dinv, row(p["gcn2"]["b"]),
        row(p["norm2"]["weight"]), row(p["norm2"]["bias"]),
        row(p["norm2"]["mean_scale"]), x2, x, wlist)

    hub_logits = hub_l[:, 0]
    return (hub_logits, hub_p[:, 0], pat_l, pat_p,
            jnp.zeros_like(hub_logits), term_l, term_p, himp[:, 0], xe)


# speedup vs baseline: 41.4901x; 41.4901x over previous
import functools

import jax
import jax.numpy as jnp
from jax import lax
from jax.experimental import pallas as pl
from jax.experimental.pallas import tpu as pltpu
from jax.experimental.pallas import tpu_sc as plsc

N = 10000
E = 320000
D = 128
H = 128
HEADS = 8
DH = 16
NPAT = 6

NC = 2
NS = 16
NW = NC * NS
LK = 128
ETOT = E + N
NB_W = 81
EW = NB_W * LK
EPAD = NW * EW
N_PAD = 10112
ROWS_T = N_PAD // NS
EPS_GN = 1e-5


def _sc_params():
    return pltpu.CompilerParams(use_tc_tiling_on_sc=False)


def _mesh():
    return plsc.VectorSubcoreMesh(
        core_axis_name="c", subcore_axis_name="s", num_cores=NC, num_subcores=NS)


def _sc_degree(cols2, ones16, zeros16):
    @functools.partial(
        pl.kernel,
        out_type=jax.ShapeDtypeStruct((NC * N_PAD, 16), jnp.float32),
        mesh=_mesh(),
        compiler_params=_sc_params(),
        scratch_types=[
            pltpu.VMEM_SHARED((N_PAD, 16), jnp.float32),
            pltpu.VMEM((NB_W, 1, LK), jnp.int32),
            pltpu.VMEM((LK, 16), jnp.float32),
        ],
    )
    def k(cols_hbm, ones_hbm, zeros_hbm, out_hbm, acc, cols_v, ones_v):
        c = lax.axis_index("c")
        s = lax.axis_index("s")
        w = c * NS + s
        pltpu.sync_copy(cols_hbm.at[w], cols_v)
        pltpu.sync_copy(ones_hbm, ones_v)
        pltpu.sync_copy(zeros_hbm, acc.at[pl.ds(s * ROWS_T, ROWS_T)])
        plsc.subcore_barrier()

        def body(nb, carry):
            pltpu.sync_copy(ones_v, acc.at[cols_v.at[nb, 0]], add=True)
            return carry

        lax.fori_loop(0, NB_W, body, 0)
        plsc.subcore_barrier()
        pltpu.sync_copy(acc.at[pl.ds(s * ROWS_T, ROWS_T)],
                        out_hbm.at[pl.ds(c * N_PAD + s * ROWS_T, ROWS_T)])

    return k(cols2, ones16, zeros16)


def _sc_gcn(hd, rows2, cols2, zeros128):
    @functools.partial(
        pl.kernel,
        out_type=jax.ShapeDtypeStruct((NC * N_PAD, H), jnp.float32),
        mesh=_mesh(),
        compiler_params=_sc_params(),
        scratch_types=[
            pltpu.VMEM_SHARED((N_PAD, H), jnp.float32),
            pltpu.VMEM((NB_W, 1, LK), jnp.int32),
            pltpu.VMEM((NB_W, 1, LK), jnp.int32),
            pltpu.VMEM((LK, H), jnp.float32),
            pltpu.SemaphoreType.DMA,
        ],
    )
    def k(hd_hbm, rows_hbm, cols_hbm, zeros_hbm, out_hbm,
          acc, rows_v, cols_v, buf, sem):
        c = lax.axis_index("c")
        s = lax.axis_index("s")
        w = c * NS + s
        pltpu.sync_copy(rows_hbm.at[w], rows_v)
        pltpu.sync_copy(cols_hbm.at[w], cols_v)
        pltpu.sync_copy(zeros_hbm, acc.at[pl.ds(s * ROWS_T, ROWS_T)])
        plsc.subcore_barrier()

        def body(nb, carry):
            pltpu.async_copy(hd_hbm.at[rows_v.at[nb, 0]], buf, sem).wait()
            pltpu.sync_copy(buf, acc.at[cols_v.at[nb, 0]], add=True)
            return carry

        lax.fori_loop(0, NB_W, body, 0)
        plsc.subcore_barrier()
        pltpu.sync_copy(acc.at[pl.ds(s * ROWS_T, ROWS_T)],
                        out_hbm.at[pl.ds(c * N_PAD + s * ROWS_T, ROWS_T)])

    return k(hd, rows2, cols2, zeros128)


def _sc_gat(hm, as16, ad16, m16, rows2, cols2, zeros128, zeros16):
    @functools.partial(
        pl.kernel,
        out_type=(jax.ShapeDtypeStruct((NC * N_PAD, H), jnp.float32),
                  jax.ShapeDtypeStruct((NC * N_PAD, 16), jnp.float32)),
        mesh=_mesh(),
        compiler_params=_sc_params(),
        scratch_types=[
            pltpu.VMEM_SHARED((N_PAD, H), jnp.float32),
            pltpu.VMEM_SHARED((N_PAD, 16), jnp.float32),
            pltpu.VMEM((2, 1, LK), jnp.int32),
            pltpu.VMEM((2, 1, LK), jnp.int32),
            pltpu.VMEM((LK, H), jnp.float32),
            pltpu.VMEM((LK, 16), jnp.float32),
            pltpu.VMEM((LK, 16), jnp.float32),
            pltpu.VMEM((16,), jnp.float32),
            pltpu.SemaphoreType.DMA,
            pltpu.SemaphoreType.DMA,
            pltpu.SemaphoreType.DMA,
            pltpu.SemaphoreType.DMA,
            pltpu.SemaphoreType.DMA,
        ],
    )
    def k(hm_hbm, as_hbm, ad_hbm, m16_hbm, rows_hbm, cols_hbm,
          z128_hbm, z16_hbm, out_hbm, den_hbm,
          acc, den, rows_v, cols_v, hbuf, asb, adb, m16v,
          sem_h, sem_a, sem_b, sem_ri, sem_ci):
        c = lax.axis_index("c")
        s = lax.axis_index("s")
        w = c * NS + s
        pltpu.async_copy(rows_hbm.at[w, 0], rows_v.at[0], sem_ri)
        pltpu.async_copy(cols_hbm.at[w, 0], cols_v.at[0], sem_ci)
        pltpu.sync_copy(m16_hbm, m16v)
        pltpu.sync_copy(z128_hbm, acc.at[pl.ds(s * ROWS_T, ROWS_T)])
        pltpu.sync_copy(z16_hbm, den.at[pl.ds(s * ROWS_T, ROWS_T)])
        plsc.subcore_barrier()

        def body(nb, carry):
            b = lax.rem(nb, 2)
            pltpu.make_async_copy(rows_hbm.at[w, nb], rows_v.at[b], sem_ri).wait()
            pltpu.make_async_copy(cols_hbm.at[w, nb], cols_v.at[b], sem_ci).wait()

            @pl.when(nb + 1 < NB_W)
            def _():
                pltpu.async_copy(rows_hbm.at[w, nb + 1], rows_v.at[1 - b],
                                 sem_ri)
                pltpu.async_copy(cols_hbm.at[w, nb + 1], cols_v.at[1 - b],
                                 sem_ci)

            cp_h = pltpu.async_copy(hm_hbm.at[rows_v.at[b, 0]], hbuf, sem_h)
            cp_a = pltpu.async_copy(as_hbm.at[rows_v.at[b, 0]], asb, sem_a)
            cp_b = pltpu.async_copy(ad_hbm.at[cols_v.at[b, 0]], adb, sem_b)
            cp_a.wait()
            cp_b.wait()
            cp_h.wait()
            m = m16v[...]

            def edge(j, carry2):
                v = asb[j] + adb[j]
                v = jnp.maximum(v, v * 0.2)
                ev = jnp.exp(v - m)
                asb[j] = ev
                for hd in range(HEADS):
                    blk = hbuf[j, pl.ds(hd * DH, DH)]
                    hbuf[j, pl.ds(hd * DH, DH)] = blk * ev[hd]
                return carry2

            lax.fori_loop(0, LK, edge, 0)
            pltpu.sync_copy(asb, den.at[cols_v.at[b, 0]], add=True)
            pltpu.sync_copy(hbuf, acc.at[cols_v.at[b, 0]], add=True)
            return carry

        lax.fori_loop(0, NB_W, body, 0)
        plsc.subcore_barrier()
        pltpu.sync_copy(acc.at[pl.ds(s * ROWS_T, ROWS_T)],
                        out_hbm.at[pl.ds(c * N_PAD + s * ROWS_T, ROWS_T)])
        pltpu.sync_copy(den.at[pl.ds(s * ROWS_T, ROWS_T)],
                        den_hbm.at[pl.ds(c * N_PAD + s * ROWS_T, ROWS_T)])

    return k(hm, as16, ad16, m16, rows2, cols2, zeros128, zeros16)


def _graphnorm(h, p_w, p_b, p_ms):
    mean = jnp.mean(h, axis=0, keepdims=True)
    xc = h - mean * p_ms
    var = jnp.mean(xc * xc, axis=0, keepdims=True)
    return p_w * xc * lax.rsqrt(var + EPS_GN) + p_b


def _tc1(x, dega, degb, we, be, w0):
    def body(x_r, dega_r, degb_r, we_r, be_r, w0_r,
             xemb_o, hd0_o, dinv_o):
        deg = dega_r[:, 0:1] + degb_r[:, 0:1]
        dinv = jnp.where(deg > 0, lax.rsqrt(deg), 0.0)
        xemb = jnp.dot(x_r[...], we_r[...],
                       preferred_element_type=jnp.float32) + be_r[...]
        hd0 = dinv * jnp.dot(xemb, w0_r[...],
                             preferred_element_type=jnp.float32)
        xemb_o[...] = xemb
        hd0_o[...] = hd0
        dinv_o[...] = dinv

    return pl.pallas_call(
        body,
        out_shape=(jax.ShapeDtypeStruct((N, H), jnp.float32),
                   jax.ShapeDtypeStruct((N, H), jnp.float32),
                   jax.ShapeDtypeStruct((N, 1), jnp.float32)),
    )(x, dega, degb, we, be, w0)


def _tc2(p0a, p0b, dinv, b0, n0w, n0b, n0ms, xemb, w1, att_s, att_d, msel):
    def body(p0a_r, p0b_r, dinv_r, b0_r, n0w_r, n0b_r, n0ms_r, xemb_r,
             w1_r, atts_r, attd_r, msel_r,
             x1_o, hm_o, as_o, ad_o, m16_o):
        h = dinv_r[...] * (p0a_r[...] + p0b_r[...]) + b0_r[...]
        h = _graphnorm(h, n0w_r[...], n0b_r[...], n0ms_r[...])
        x1 = jnp.maximum(h, 0.0) + xemb_r[...]
        hm = jnp.dot(x1, w1_r[...], preferred_element_type=jnp.float32)
        a_s = jnp.dot(hm * atts_r[...], msel_r[...],
                      preferred_element_type=jnp.float32)
        a_d = jnp.dot(hm * attd_r[...], msel_r[...],
                      preferred_element_type=jnp.float32)
        mm = (jnp.max(a_s, axis=0, keepdims=True)
              + jnp.max(a_d, axis=0, keepdims=True))
        mub = jnp.maximum(mm, mm * 0.2)
        zero8 = jnp.zeros((N, 8), jnp.float32)
        x1_o[...] = x1
        hm_o[...] = hm
        as_o[...] = jnp.concatenate([a_s, zero8], axis=1)
        ad_o[...] = jnp.concatenate([a_d, zero8], axis=1)
        m16_o[...] = jnp.concatenate([mub, jnp.zeros((1, 8), jnp.float32)],
                                     axis=1)

    return pl.pallas_call(
        body,
        out_shape=(jax.ShapeDtypeStruct((N, H), jnp.float32),
                   jax.ShapeDtypeStruct((N, H), jnp.float32),
                   jax.ShapeDtypeStruct((N, 16), jnp.float32),
                   jax.ShapeDtypeStruct((N, 16), jnp.float32),
                   jax.ShapeDtypeStruct((1, 16), jnp.float32)),
    )(p0a, p0b, dinv, b0, n0w, n0b, n0ms, xemb, w1, att_s, att_d, msel)


def _tc4(paa, pab, dena, denb, b1, n1w, n1b, n1ms, x1, w2, dinv, mexp):
    def body(paa_r, pab_r, dena_r, denb_r, b1_r, n1w_r, n1b_r, n1ms_r,
             x1_r, w2_r, dinv_r, mexp_r, x2_o, hd2_o):
        den = dena_r[:, 0:8] + denb_r[:, 0:8]
        scale = 1.0 / (den + 1e-16)
        scale128 = jnp.dot(scale, mexp_r[...],
                           preferred_element_type=jnp.float32)
        h = (paa_r[...] + pab_r[...]) * scale128 + b1_r[...]
        h = _graphnorm(h, n1w_r[...], n1b_r[...], n1ms_r[...])
        x2 = jnp.maximum(h, 0.0) + x1_r[...]
        hd2 = dinv_r[...] * jnp.dot(x2, w2_r[...],
                                    preferred_element_type=jnp.float32)
        x2_o[...] = x2
        hd2_o[...] = hd2

    return pl.pallas_call(
        body,
        out_shape=(jax.ShapeDtypeStruct((N, H), jnp.float32),
                   jax.ShapeDtypeStruct((N, H), jnp.float32)),
    )(paa, pab, dena, denb, b1, n1w, n1b, n1ms, x1, w2, dinv, mexp)


def _tc5(p2a, p2b, dinv, b2, n2w, n2b, n2ms, x2, x, wlist):
    def body(p2a_r, p2b_r, dinv_r, b2_r, n2w_r, n2b_r, n2ms_r, x2_r, x_r,
             wi1x_r, wi1s_r, bi1_r, wi2_r, bi2_r,
             ws1x_r, ws1h_r, bs1_r, ws2_r, bs2_r,
             wp1x_r, wp1gm_r, wp1gx_r, wp1s_r, bp1_r,
             wp2_r, bp2_r, wp3_r, bp3_r,
             wt1m_r, wt1x_r, bt1_r, wt2_r, bt2_r,
             hl_o, hp_o, plg_o, pp_o, tl_o, tp_o, hi_o, xe_o):
        h = dinv_r[...] * (p2a_r[...] + p2b_r[...]) + b2_r[...]
        h = _graphnorm(h, n2w_r[...], n2b_r[...], n2ms_r[...])
        xe = jnp.maximum(h, 0.0) + x2_r[...]
        structural = x_r[:, 0:6]

        hi1 = jnp.dot(xe, wi1x_r[...], preferred_element_type=jnp.float32)
        hi1 = hi1 + jnp.dot(structural, wi1s_r[...],
                            preferred_element_type=jnp.float32) + bi1_r[...]
        hi1 = jnp.maximum(hi1, 0.0)
        himp_l = jnp.dot(hi1, wi2_r[...],
                         preferred_element_type=jnp.float32) + bi2_r[...]
        himp = 1.0 / (1.0 + jnp.exp(-himp_l))

        hs1 = jnp.dot(xe, ws1x_r[...], preferred_element_type=jnp.float32)
        hs1 = hs1 + himp * ws1h_r[...] + bs1_r[...]
        hs1 = jnp.maximum(hs1, 0.0)
        hub_l = (jnp.dot(hs1, ws2_r[...], preferred_element_type=jnp.float32)
                 + bs2_r[...] + 2.0 * himp)
        mx = jnp.max(hub_l, axis=0, keepdims=True)
        ex = jnp.exp(hub_l - mx)
        hub_p = ex / jnp.sum(ex, axis=0, keepdims=True)

        gmean = jnp.mean(xe, axis=0, keepdims=True)
        gmax = jnp.max(xe, axis=0, keepdims=True)

        pat = jnp.dot(xe, wp1x_r[...], preferred_element_type=jnp.float32)
        gfp = (jnp.dot(gmean, wp1gm_r[...], preferred_element_type=jnp.float32)
               + jnp.dot(gmax, wp1gx_r[...], preferred_element_type=jnp.float32))
        pat = pat + gfp + jnp.dot(structural, wp1s_r[...],
                                  preferred_element_type=jnp.float32) + bp1_r[...]
        pat = jnp.maximum(pat, 0.0)
        pat = jnp.maximum(jnp.dot(pat, wp2_r[...],
                                  preferred_element_type=jnp.float32)
                          + bp2_r[...], 0.0)
        pat_l = jnp.dot(pat, wp3_r[...],
                        preferred_element_type=jnp.float32) + bp3_r[...]
        pm = jnp.max(pat_l, axis=1, keepdims=True)
        pe = jnp.exp(pat_l - pm)
        pat_p = pe / jnp.sum(pe, axis=1, keepdims=True)

        t = (jnp.dot(gmean, wt1m_r[...], preferred_element_type=jnp.float32)
             + jnp.dot(gmax, wt1x_r[...], preferred_element_type=jnp.float32)
             + bt1_r[...])
        t = jnp.maximum(t, 0.0)
        term_l = jnp.dot(t, wt2_r[...],
                         preferred_element_type=jnp.float32) + bt2_r[...]
        tm = jnp.max(term_l, axis=1, keepdims=True)
        te = jnp.exp(term_l - tm)
        term_p = te / jnp.sum(te, axis=1, keepdims=True)

        hl_o[...] = hub_l
        hp_o[...] = hub_p
        plg_o[...] = pat_l
        pp_o[...] = pat_p
        tl_o[...] = term_l
        tp_o[...] = term_p
        hi_o[...] = himp
        xe_o[...] = xe

    return pl.pallas_call(
        body,
        out_shape=(jax.ShapeDtypeStruct((N, 1), jnp.float32),
                   jax.ShapeDtypeStruct((N, 1), jnp.float32),
                   jax.ShapeDtypeStruct((N, NPAT), jnp.float32),
                   jax.ShapeDtypeStruct((N, NPAT), jnp.float32),
                   jax.ShapeDtypeStruct((1, 2), jnp.float32),
                   jax.ShapeDtypeStruct((1, 2), jnp.float32),
                   jax.ShapeDtypeStruct((N, 1), jnp.float32),
                   jax.ShapeDtypeStruct((N, H), jnp.float32)),
        compiler_params=pltpu.CompilerParams(
            vmem_limit_bytes=100 * 1024 * 1024),
    )(p2a, p2b, dinv, b2, n2w, n2b, n2ms, x2, x, *wlist)


def kernel(x, edge_index, batch, params):
    del batch
    f32 = jnp.float32

    loops = jnp.arange(N, dtype=jnp.int32)
    pad = EPAD - ETOT
    rows = jnp.concatenate([edge_index[0], loops,
                            jnp.zeros((pad,), jnp.int32)])
    cols = jnp.concatenate([edge_index[1], loops,
                            jnp.full((pad,), N, jnp.int32)])
    rows2 = rows.reshape(NW, NB_W, 1, LK)
    cols2 = cols.reshape(NW, NB_W, 1, LK)

    zeros128 = jnp.zeros((ROWS_T, H), f32)
    zeros16 = jnp.zeros((ROWS_T, 16), f32)
    ones16 = jnp.ones((LK, 16), f32)

    msel = jnp.repeat(jnp.eye(HEADS, dtype=f32), DH, axis=0)
    mexp = msel.T

    p = params
    row = lambda v: v.reshape(1, -1)

    degout = _sc_degree(cols2, ones16, zeros16)
    dega = degout[0:N]
    degb = degout[N_PAD:N_PAD + N]

    xemb, hd0, dinv = _tc1(x, dega, degb,
                           p["node_embed"]["w"], row(p["node_embed"]["b"]),
                           p["gcn0"]["w"])

    p0 = _sc_gcn(hd0, rows2, cols2, zeros128)
    p0a, p0b = p0[0:N], p0[N_PAD:N_PAD + N]

    x1, hm, as16, ad16, m16 = _tc2(
        p0a, p0b, dinv, row(p["gcn0"]["b"]),
        row(p["norm0"]["weight"]), row(p["norm0"]["bias"]),
        row(p["norm0"]["mean_scale"]), xemb,
        p["gat1"]["w"], row(p["gat1"]["att_src"].reshape(-1)),
        row(p["gat1"]["att_dst"].reshape(-1)), msel)

    pa, denp = _sc_gat(hm, as16, ad16, m16.reshape(16), rows2, cols2,
                       zeros128, zeros16)
    paa, pab = pa[0:N], pa[N_PAD:N_PAD + N]
    dena, denb = denp[0:N], denp[N_PAD:N_PAD + N]

    x2, hd2 = _tc4(paa, pab, dena, denb, row(p["gat1"]["b"]),
                   row(p["norm1"]["weight"]), row(p["norm1"]["bias"]),
                   row(p["norm1"]["mean_scale"]), x1, p["gcn2"]["w"],
                   dinv, mexp)

    p2 = _sc_gcn(hd2, rows2, cols2, zeros128)
    p2a, p2b = p2[0:N], p2[N_PAD:N_PAD + N]

    wi1 = p["hub_imp1"]["w"]
    ws1 = p["hub_sel1"]["w"]
    pw1 = p["pat1"]["w"]
    tw1 = p["term1"]["w"]
    wlist = (
        wi1[0:H], wi1[H:H + 6], row(p["hub_imp1"]["b"]),
        p["hub_imp2"]["w"], row(p["hub_imp2"]["b"]),
        ws1[0:H], row(ws1[H]), row(p["hub_sel1"]["b"]),
        p["hub_sel2"]["w"], row(p["hub_sel2"]["b"]),
        pw1[0:H], pw1[H:2 * H], pw1[2 * H:3 * H], pw1[3 * H:3 * H + 6],
        row(p["pat1"]["b"]),
        p["pat2"]["w"], row(p["pat2"]["b"]),
        p["pat3"]["w"], row(p["pat3"]["b"]),
        tw1[0:H], tw1[H:2 * H], row(p["term1"]["b"]),
        p["term2"]["w"], row(p["term2"]["b"]),
    )
    (hub_l, hub_p, pat_l, pat_p, term_l, term_p, himp, xe) = _tc5(
        p2a, p2b, dinv, row(p["gcn2"]["b"]),
        row(p["norm2"]["weight"]), row(p["norm2"]["bias"]),
        row(p["norm2"]["mean_scale"]), x2, x, wlist)

    hub_logits = hub_l[:, 0]
    return (hub_logits, hub_p[:, 0], pat_l, pat_p,
            jnp.zeros_like(hub_logits), term_l, term_p, himp[:, 0], xe)


# speedup vs baseline: 47.2594x; 1.1391x over previous
import functools

import jax
import jax.numpy as jnp
from jax import lax
from jax.experimental import pallas as pl
from jax.experimental.pallas import tpu as pltpu
from jax.experimental.pallas import tpu_sc as plsc

N = 10000
E = 320000
D = 128
H = 128
HEADS = 8
DH = 16
NPAT = 6

NC = 2
NS = 16
NW = NC * NS
LK = 128
ETOT = E + N
NB_W = 81
EW = NB_W * LK
EPAD = NW * EW
N_PAD = 10112
ROWS_T = N_PAD // NS
LKG = 64
NBG = EW // LKG
EPS_GN = 1e-5


def _sc_params():
    return pltpu.CompilerParams(use_tc_tiling_on_sc=False)


def _mesh():
    return plsc.VectorSubcoreMesh(
        core_axis_name="c", subcore_axis_name="s", num_cores=NC, num_subcores=NS)


def _sc_degree(cols2, ones16, zeros16):
    @functools.partial(
        pl.kernel,
        out_type=jax.ShapeDtypeStruct((NC * N_PAD, 16), jnp.float32),
        mesh=_mesh(),
        compiler_params=_sc_params(),
        scratch_types=[
            pltpu.VMEM_SHARED((N_PAD, 16), jnp.float32),
            pltpu.VMEM((NB_W, 1, LK), jnp.int32),
            pltpu.VMEM((LK, 16), jnp.float32),
        ],
    )
    def k(cols_hbm, ones_hbm, zeros_hbm, out_hbm, acc, cols_v, ones_v):
        c = lax.axis_index("c")
        s = lax.axis_index("s")
        w = c * NS + s
        pltpu.sync_copy(cols_hbm.at[w], cols_v)
        pltpu.sync_copy(ones_hbm, ones_v)
        pltpu.sync_copy(zeros_hbm, acc.at[pl.ds(s * ROWS_T, ROWS_T)])
        plsc.subcore_barrier()

        def body(nb, carry):
            pltpu.sync_copy(ones_v, acc.at[cols_v.at[nb, 0]], add=True)
            return carry

        lax.fori_loop(0, NB_W, body, 0)
        plsc.subcore_barrier()
        pltpu.sync_copy(acc.at[pl.ds(s * ROWS_T, ROWS_T)],
                        out_hbm.at[pl.ds(c * N_PAD + s * ROWS_T, ROWS_T)])

    return k(cols2, ones16, zeros16)


def _sc_gcn(hd, rows2, cols2, zeros128):
    @functools.partial(
        pl.kernel,
        out_type=jax.ShapeDtypeStruct((NC * N_PAD, H), jnp.float32),
        mesh=_mesh(),
        compiler_params=_sc_params(),
        scratch_types=[
            pltpu.VMEM_SHARED((N_PAD, H), jnp.float32),
            pltpu.VMEM((2, 1, LK), jnp.int32),
            pltpu.VMEM((2, 1, LK), jnp.int32),
            pltpu.VMEM((2 * LK, H), jnp.float32),
            pltpu.SemaphoreType.DMA,
            pltpu.SemaphoreType.DMA,
            pltpu.SemaphoreType.DMA,
        ],
    )
    def k(hd_hbm, rows_hbm, cols_hbm, zeros_hbm, out_hbm,
          acc, rows_v, cols_v, buf, sem_g, sem_ri, sem_ci):
        c = lax.axis_index("c")
        s = lax.axis_index("s")
        w = c * NS + s
        pltpu.async_copy(rows_hbm.at[w, 0], rows_v.at[0], sem_ri)
        pltpu.async_copy(cols_hbm.at[w, 0], cols_v.at[0], sem_ci)
        pltpu.sync_copy(zeros_hbm, acc.at[pl.ds(s * ROWS_T, ROWS_T)])
        plsc.subcore_barrier()
        pltpu.make_async_copy(rows_hbm.at[w, 0], rows_v.at[0], sem_ri).wait()
        pltpu.make_async_copy(cols_hbm.at[w, 0], cols_v.at[0], sem_ci).wait()
        pltpu.async_copy(hd_hbm.at[rows_v.at[0, 0]], buf.at[pl.ds(0, LK)],
                         sem_g)
        pltpu.async_copy(rows_hbm.at[w, 1], rows_v.at[1], sem_ri)
        pltpu.async_copy(cols_hbm.at[w, 1], cols_v.at[1], sem_ci)

        def body(nb, carry):
            b = lax.rem(nb, 2)
            pltpu.make_async_copy(hd_hbm.at[rows_v.at[b, 0]],
                                  buf.at[pl.ds(b * LK, LK)], sem_g).wait()

            @pl.when(nb + 1 < NB_W)
            def _():
                pltpu.make_async_copy(rows_hbm.at[w, nb + 1],
                                      rows_v.at[1 - b], sem_ri).wait()
                pltpu.make_async_copy(cols_hbm.at[w, nb + 1],
                                      cols_v.at[1 - b], sem_ci).wait()
                pltpu.async_copy(hd_hbm.at[rows_v.at[1 - b, 0]],
                                 buf.at[pl.ds((1 - b) * LK, LK)], sem_g)

            pltpu.sync_copy(buf.at[pl.ds(b * LK, LK)],
                            acc.at[cols_v.at[b, 0]], add=True)

            @pl.when(nb + 2 < NB_W)
            def _():
                pltpu.async_copy(rows_hbm.at[w, nb + 2], rows_v.at[b], sem_ri)
                pltpu.async_copy(cols_hbm.at[w, nb + 2], cols_v.at[b], sem_ci)

            return carry

        lax.fori_loop(0, NB_W, body, 0)
        plsc.subcore_barrier()
        pltpu.sync_copy(acc.at[pl.ds(s * ROWS_T, ROWS_T)],
                        out_hbm.at[pl.ds(c * N_PAD + s * ROWS_T, ROWS_T)])

    return k(hd, rows2, cols2, zeros128)


def _sc_gat(hm, as16, ad16, m16, rows2g, cols2g, zeros128, zeros16):
    @functools.partial(
        pl.kernel,
        out_type=(jax.ShapeDtypeStruct((NC * N_PAD, H), jnp.float32),
                  jax.ShapeDtypeStruct((NC * N_PAD, 16), jnp.float32)),
        mesh=_mesh(),
        compiler_params=_sc_params(),
        scratch_types=[
            pltpu.VMEM_SHARED((N_PAD, H), jnp.float32),
            pltpu.VMEM_SHARED((N_PAD, 16), jnp.float32),
            pltpu.VMEM((2, 1, LKG), jnp.int32),
            pltpu.VMEM((2, 1, LKG), jnp.int32),
            pltpu.VMEM((2 * LKG, H), jnp.float32),
            pltpu.VMEM((2 * LKG, 16), jnp.float32),
            pltpu.VMEM((2 * LKG, 16), jnp.float32),
            pltpu.VMEM((16,), jnp.float32),
            pltpu.SemaphoreType.DMA,
            pltpu.SemaphoreType.DMA,
            pltpu.SemaphoreType.DMA,
            pltpu.SemaphoreType.DMA,
            pltpu.SemaphoreType.DMA,
        ],
    )
    def k(hm_hbm, as_hbm, ad_hbm, m16_hbm, rows_hbm, cols_hbm,
          z128_hbm, z16_hbm, out_hbm, den_hbm,
          acc, den, rows_v, cols_v, hbuf, asb, adb, m16v,
          sem_h, sem_a, sem_b, sem_ri, sem_ci):
        c = lax.axis_index("c")
        s = lax.axis_index("s")
        w = c * NS + s
        pltpu.async_copy(rows_hbm.at[w, 0], rows_v.at[0], sem_ri)
        pltpu.async_copy(cols_hbm.at[w, 0], cols_v.at[0], sem_ci)
        pltpu.sync_copy(m16_hbm, m16v)
        pltpu.sync_copy(z128_hbm, acc.at[pl.ds(s * ROWS_T, ROWS_T)])
        pltpu.sync_copy(z16_hbm, den.at[pl.ds(s * ROWS_T, ROWS_T)])
        plsc.subcore_barrier()

        def gathers(src_b, dst_b):
            pltpu.async_copy(hm_hbm.at[rows_v.at[src_b, 0]],
                             hbuf.at[pl.ds(dst_b * LKG, LKG)], sem_h)
            pltpu.async_copy(as_hbm.at[rows_v.at[src_b, 0]],
                             asb.at[pl.ds(dst_b * LKG, LKG)], sem_a)
            pltpu.async_copy(ad_hbm.at[cols_v.at[src_b, 0]],
                             adb.at[pl.ds(dst_b * LKG, LKG)], sem_b)

        def wait_gathers(src_b, dst_b):
            pltpu.make_async_copy(hm_hbm.at[rows_v.at[src_b, 0]],
                                  hbuf.at[pl.ds(dst_b * LKG, LKG)],
                                  sem_h).wait()
            pltpu.make_async_copy(as_hbm.at[rows_v.at[src_b, 0]],
                                  asb.at[pl.ds(dst_b * LKG, LKG)],
                                  sem_a).wait()
            pltpu.make_async_copy(ad_hbm.at[cols_v.at[src_b, 0]],
                                  adb.at[pl.ds(dst_b * LKG, LKG)],
                                  sem_b).wait()

        pltpu.make_async_copy(rows_hbm.at[w, 0], rows_v.at[0], sem_ri).wait()
        pltpu.make_async_copy(cols_hbm.at[w, 0], cols_v.at[0], sem_ci).wait()
        gathers(0, 0)
        pltpu.async_copy(rows_hbm.at[w, 1], rows_v.at[1], sem_ri)
        pltpu.async_copy(cols_hbm.at[w, 1], cols_v.at[1], sem_ci)
        m = m16v[...]

        def body(nb, carry):
            b = lax.rem(nb, 2)
            wait_gathers(b, b)

            @pl.when(nb + 1 < NBG)
            def _():
                pltpu.make_async_copy(rows_hbm.at[w, nb + 1],
                                      rows_v.at[1 - b], sem_ri).wait()
                pltpu.make_async_copy(cols_hbm.at[w, nb + 1],
                                      cols_v.at[1 - b], sem_ci).wait()
                gathers(1 - b, 1 - b)

            base = b * LKG

            def edge(j, carry2):
                v = asb[base + j] + adb[base + j]
                v = jnp.maximum(v, v * 0.2)
                ev = jnp.exp(v - m)
                asb[base + j] = ev
                for hd in range(HEADS):
                    blk = hbuf[base + j, pl.ds(hd * DH, DH)]
                    hbuf[base + j, pl.ds(hd * DH, DH)] = blk * ev[hd]
                return carry2

            lax.fori_loop(0, LKG, edge, 0)
            pltpu.sync_copy(asb.at[pl.ds(base, LKG)],
                            den.at[cols_v.at[b, 0]], add=True)
            pltpu.sync_copy(hbuf.at[pl.ds(base, LKG)],
                            acc.at[cols_v.at[b, 0]], add=True)

            @pl.when(nb + 2 < NBG)
            def _():
                pltpu.async_copy(rows_hbm.at[w, nb + 2], rows_v.at[b], sem_ri)
                pltpu.async_copy(cols_hbm.at[w, nb + 2], cols_v.at[b], sem_ci)

            return carry

        lax.fori_loop(0, NBG, body, 0)
        plsc.subcore_barrier()
        pltpu.sync_copy(acc.at[pl.ds(s * ROWS_T, ROWS_T)],
                        out_hbm.at[pl.ds(c * N_PAD + s * ROWS_T, ROWS_T)])
        pltpu.sync_copy(den.at[pl.ds(s * ROWS_T, ROWS_T)],
                        den_hbm.at[pl.ds(c * N_PAD + s * ROWS_T, ROWS_T)])

    return k(hm, as16, ad16, m16, rows2g, cols2g, zeros128, zeros16)


def _graphnorm(h, p_w, p_b, p_ms):
    mean = jnp.mean(h, axis=0, keepdims=True)
    xc = h - mean * p_ms
    var = jnp.mean(xc * xc, axis=0, keepdims=True)
    return p_w * xc * lax.rsqrt(var + EPS_GN) + p_b


def _tc1(x, dega, degb, we, be, w0):
    def body(x_r, dega_r, degb_r, we_r, be_r, w0_r,
             xemb_o, hd0_o, dinv_o):
        deg = dega_r[:, 0:1] + degb_r[:, 0:1]
        dinv = jnp.where(deg > 0, lax.rsqrt(deg), 0.0)
        xemb = jnp.dot(x_r[...], we_r[...],
                       preferred_element_type=jnp.float32) + be_r[...]
        hd0 = dinv * jnp.dot(xemb, w0_r[...],
                             preferred_element_type=jnp.float32)
        xemb_o[...] = xemb
        hd0_o[...] = hd0
        dinv_o[...] = dinv

    return pl.pallas_call(
        body,
        out_shape=(jax.ShapeDtypeStruct((N, H), jnp.float32),
                   jax.ShapeDtypeStruct((N, H), jnp.float32),
                   jax.ShapeDtypeStruct((N, 1), jnp.float32)),
    )(x, dega, degb, we, be, w0)


def _tc2(p0a, p0b, dinv, b0, n0w, n0b, n0ms, xemb, w1, att_s, att_d, msel):
    def body(p0a_r, p0b_r, dinv_r, b0_r, n0w_r, n0b_r, n0ms_r, xemb_r,
             w1_r, atts_r, attd_r, msel_r,
             x1_o, hm_o, as_o, ad_o, m16_o):
        h = dinv_r[...] * (p0a_r[...] + p0b_r[...]) + b0_r[...]
        h = _graphnorm(h, n0w_r[...], n0b_r[...], n0ms_r[...])
        x1 = jnp.maximum(h, 0.0) + xemb_r[...]
        hm = jnp.dot(x1, w1_r[...], preferred_element_type=jnp.float32)
        a_s = jnp.dot(hm * atts_r[...], msel_r[...],
                      preferred_element_type=jnp.float32)
        a_d = jnp.dot(hm * attd_r[...], msel_r[...],
                      preferred_element_type=jnp.float32)
        mm = (jnp.max(a_s, axis=0, keepdims=True)
              + jnp.max(a_d, axis=0, keepdims=True))
        mub = jnp.maximum(mm, mm * 0.2)
        zero8 = jnp.zeros((N, 8), jnp.float32)
        x1_o[...] = x1
        hm_o[...] = hm
        as_o[...] = jnp.concatenate([a_s, zero8], axis=1)
        ad_o[...] = jnp.concatenate([a_d, zero8], axis=1)
        m16_o[...] = jnp.concatenate([mub, jnp.zeros((1, 8), jnp.float32)],
                                     axis=1)

    return pl.pallas_call(
        body,
        out_shape=(jax.ShapeDtypeStruct((N, H), jnp.float32),
                   jax.ShapeDtypeStruct((N, H), jnp.float32),
                   jax.ShapeDtypeStruct((N, 16), jnp.float32),
                   jax.ShapeDtypeStruct((N, 16), jnp.float32),
                   jax.ShapeDtypeStruct((1, 16), jnp.float32)),
    )(p0a, p0b, dinv, b0, n0w, n0b, n0ms, xemb, w1, att_s, att_d, msel)


def _tc4(paa, pab, dena, denb, b1, n1w, n1b, n1ms, x1, w2, dinv, mexp):
    def body(paa_r, pab_r, dena_r, denb_r, b1_r, n1w_r, n1b_r, n1ms_r,
             x1_r, w2_r, dinv_r, mexp_r, x2_o, hd2_o):
        den = dena_r[:, 0:8] + denb_r[:, 0:8]
        scale = 1.0 / (den + 1e-16)
        scale128 = jnp.dot(scale, mexp_r[...],
                           preferred_element_type=jnp.float32)
        h = (paa_r[...] + pab_r[...]) * scale128 + b1_r[...]
        h = _graphnorm(h, n1w_r[...], n1b_r[...], n1ms_r[...])
        x2 = jnp.maximum(h, 0.0) + x1_r[...]
        hd2 = dinv_r[...] * jnp.dot(x2, w2_r[...],
                                    preferred_element_type=jnp.float32)
        x2_o[...] = x2
        hd2_o[...] = hd2

    return pl.pallas_call(
        body,
        out_shape=(jax.ShapeDtypeStruct((N, H), jnp.float32),
                   jax.ShapeDtypeStruct((N, H), jnp.float32)),
    )(paa, pab, dena, denb, b1, n1w, n1b, n1ms, x1, w2, dinv, mexp)


def _tc5(p2a, p2b, dinv, b2, n2w, n2b, n2ms, x2, x, wlist):
    def body(p2a_r, p2b_r, dinv_r, b2_r, n2w_r, n2b_r, n2ms_r, x2_r, x_r,
             wi1x_r, wi1s_r, bi1_r, wi2_r, bi2_r,
             ws1x_r, ws1h_r, bs1_r, ws2_r, bs2_r,
             wp1x_r, wp1gm_r, wp1gx_r, wp1s_r, bp1_r,
             wp2_r, bp2_r, wp3_r, bp3_r,
             wt1m_r, wt1x_r, bt1_r, wt2_r, bt2_r,
             hl_o, hp_o, plg_o, pp_o, tl_o, tp_o, hi_o, xe_o):
        h = dinv_r[...] * (p2a_r[...] + p2b_r[...]) + b2_r[...]
        h = _graphnorm(h, n2w_r[...], n2b_r[...], n2ms_r[...])
        xe = jnp.maximum(h, 0.0) + x2_r[...]
        structural = x_r[:, 0:6]

        hi1 = jnp.dot(xe, wi1x_r[...], preferred_element_type=jnp.float32)
        hi1 = hi1 + jnp.dot(structural, wi1s_r[...],
                            preferred_element_type=jnp.float32) + bi1_r[...]
        hi1 = jnp.maximum(hi1, 0.0)
        himp_l = jnp.dot(hi1, wi2_r[...],
                         preferred_element_type=jnp.float32) + bi2_r[...]
        himp = 1.0 / (1.0 + jnp.exp(-himp_l))

        hs1 = jnp.dot(xe, ws1x_r[...], preferred_element_type=jnp.float32)
        hs1 = hs1 + himp * ws1h_r[...] + bs1_r[...]
        hs1 = jnp.maximum(hs1, 0.0)
        hub_l = (jnp.dot(hs1, ws2_r[...], preferred_element_type=jnp.float32)
                 + bs2_r[...] + 2.0 * himp)
        mx = jnp.max(hub_l, axis=0, keepdims=True)
        ex = jnp.exp(hub_l - mx)
        hub_p = ex / jnp.sum(ex, axis=0, keepdims=True)

        gmean = jnp.mean(xe, axis=0, keepdims=True)
        gmax = jnp.max(xe, axis=0, keepdims=True)

        pat = jnp.dot(xe, wp1x_r[...], preferred_element_type=jnp.float32)
        gfp = (jnp.dot(gmean, wp1gm_r[...], preferred_element_type=jnp.float32)
               + jnp.dot(gmax, wp1gx_r[...], preferred_element_type=jnp.float32))
        pat = pat + gfp + jnp.dot(structural, wp1s_r[...],
                                  preferred_element_type=jnp.float32) + bp1_r[...]
        pat = jnp.maximum(pat, 0.0)
        pat = jnp.maximum(jnp.dot(pat, wp2_r[...],
                                  preferred_element_type=jnp.float32)
                          + bp2_r[...], 0.0)
        pat_l = jnp.dot(pat, wp3_r[...],
                        preferred_element_type=jnp.float32) + bp3_r[...]
        pm = jnp.max(pat_l, axis=1, keepdims=True)
        pe = jnp.exp(pat_l - pm)
        pat_p = pe / jnp.sum(pe, axis=1, keepdims=True)

        t = (jnp.dot(gmean, wt1m_r[...], preferred_element_type=jnp.float32)
             + jnp.dot(gmax, wt1x_r[...], preferred_element_type=jnp.float32)
             + bt1_r[...])
        t = jnp.maximum(t, 0.0)
        term_l = jnp.dot(t, wt2_r[...],
                         preferred_element_type=jnp.float32) + bt2_r[...]
        tm = jnp.max(term_l, axis=1, keepdims=True)
        te = jnp.exp(term_l - tm)
        term_p = te / jnp.sum(te, axis=1, keepdims=True)

        hl_o[...] = hub_l
        hp_o[...] = hub_p
        plg_o[...] = pat_l
        pp_o[...] = pat_p
        tl_o[...] = term_l
        tp_o[...] = term_p
        hi_o[...] = himp
        xe_o[...] = xe

    return pl.pallas_call(
        body,
        out_shape=(jax.ShapeDtypeStruct((N, 1), jnp.float32),
                   jax.ShapeDtypeStruct((N, 1), jnp.float32),
                   jax.ShapeDtypeStruct((N, NPAT), jnp.float32),
                   jax.ShapeDtypeStruct((N, NPAT), jnp.float32),
                   jax.ShapeDtypeStruct((1, 2), jnp.float32),
                   jax.ShapeDtypeStruct((1, 2), jnp.float32),
                   jax.ShapeDtypeStruct((N, 1), jnp.float32),
                   jax.ShapeDtypeStruct((N, H), jnp.float32)),
        compiler_params=pltpu.CompilerParams(
            vmem_limit_bytes=100 * 1024 * 1024),
    )(p2a, p2b, dinv, b2, n2w, n2b, n2ms, x2, x, *wlist)


def kernel(x, edge_index, batch, params):
    del batch
    f32 = jnp.float32

    loops = jnp.arange(N, dtype=jnp.int32)
    pad = EPAD - ETOT
    rows = jnp.concatenate([edge_index[0], loops,
                            jnp.zeros((pad,), jnp.int32)])
    cols = jnp.concatenate([edge_index[1], loops,
                            jnp.full((pad,), N, jnp.int32)])
    rows2 = rows.reshape(NW, NB_W, 1, LK)
    cols2 = cols.reshape(NW, NB_W, 1, LK)
    rows2g = rows.reshape(NW, NBG, 1, LKG)
    cols2g = cols.reshape(NW, NBG, 1, LKG)

    zeros128 = jnp.zeros((ROWS_T, H), f32)
    zeros16 = jnp.zeros((ROWS_T, 16), f32)
    ones16 = jnp.ones((LK, 16), f32)

    msel = jnp.repeat(jnp.eye(HEADS, dtype=f32), DH, axis=0)
    mexp = msel.T

    p = params
    row = lambda v: v.reshape(1, -1)

    degout = _sc_degree(cols2, ones16, zeros16)
    dega = degout[0:N]
    degb = degout[N_PAD:N_PAD + N]

    xemb, hd0, dinv = _tc1(x, dega, degb,
                           p["node_embed"]["w"], row(p["node_embed"]["b"]),
                           p["gcn0"]["w"])

    p0 = _sc_gcn(hd0, rows2, cols2, zeros128)
    p0a, p0b = p0[0:N], p0[N_PAD:N_PAD + N]

    x1, hm, as16, ad16, m16 = _tc2(
        p0a, p0b, dinv, row(p["gcn0"]["b"]),
        row(p["norm0"]["weight"]), row(p["norm0"]["bias"]),
        row(p["norm0"]["mean_scale"]), xemb,
        p["gat1"]["w"], row(p["gat1"]["att_src"].reshape(-1)),
        row(p["gat1"]["att_dst"].reshape(-1)), msel)

    pa, denp = _sc_gat(hm, as16, ad16, m16.reshape(16), rows2g, cols2g,
                       zeros128, zeros16)
    paa, pab = pa[0:N], pa[N_PAD:N_PAD + N]
    dena, denb = denp[0:N], denp[N_PAD:N_PAD + N]

    x2, hd2 = _tc4(paa, pab, dena, denb, row(p["gat1"]["b"]),
                   row(p["norm1"]["weight"]), row(p["norm1"]["bias"]),
                   row(p["norm1"]["mean_scale"]), x1, p["gcn2"]["w"],
                   dinv, mexp)

    p2 = _sc_gcn(hd2, rows2, cols2, zeros128)
    p2a, p2b = p2[0:N], p2[N_PAD:N_PAD + N]

    wi1 = p["hub_imp1"]["w"]
    ws1 = p["hub_sel1"]["w"]
    pw1 = p["pat1"]["w"]
    tw1 = p["term1"]["w"]
    wlist = (
        wi1[0:H], wi1[H:H + 6], row(p["hub_imp1"]["b"]),
        p["hub_imp2"]["w"], row(p["hub_imp2"]["b"]),
        ws1[0:H], row(ws1[H]), row(p["hub_sel1"]["b"]),
        p["hub_sel2"]["w"], row(p["hub_sel2"]["b"]),
        pw1[0:H], pw1[H:2 * H], pw1[2 * H:3 * H], pw1[3 * H:3 * H + 6],
        row(p["pat1"]["b"]),
        p["pat2"]["w"], row(p["pat2"]["b"]),
        p["pat3"]["w"], row(p["pat3"]["b"]),
        tw1[0:H], tw1[H:2 * H], row(p["term1"]["b"]),
        p["term2"]["w"], row(p["term2"]["b"]),
    )
    (hub_l, hub_p, pat_l, pat_p, term_l, term_p, himp, xe) = _tc5(
        p2a, p2b, dinv, row(p["gcn2"]["b"]),
        row(p["norm2"]["weight"]), row(p["norm2"]["bias"]),
        row(p["norm2"]["mean_scale"]), x2, x, wlist)

    hub_logits = hub_l[:, 0]
    return (hub_logits, hub_p[:, 0], pat_l, pat_p,
            jnp.zeros_like(hub_logits), term_l, term_p, himp[:, 0], xe)


# speedup vs baseline: 57.1433x; 1.2091x over previous
import functools

import jax
import jax.numpy as jnp
from jax import lax
from jax.experimental import pallas as pl
from jax.experimental.pallas import tpu as pltpu
from jax.experimental.pallas import tpu_sc as plsc

N = 10000
E = 320000
D = 128
H = 128
HEADS = 8
DH = 16
NPAT = 6

NC = 2
NS = 16
NW = NC * NS
LK = 128
ETOT = E + N
NB_W = 81
EW = NB_W * LK
EPAD = NW * EW
N_PAD = 10112
ROWS_T = N_PAD // NS
LKG = 64
NBG = EW // LKG
EPS_GN = 1e-5


def _sc_params():
    return pltpu.CompilerParams(use_tc_tiling_on_sc=False)


def _mesh():
    return plsc.VectorSubcoreMesh(
        core_axis_name="c", subcore_axis_name="s", num_cores=NC, num_subcores=NS)


def _sc_degree(cols2, ones16, zeros16):
    @functools.partial(
        pl.kernel,
        out_type=jax.ShapeDtypeStruct((NC * N_PAD, 16), jnp.float32),
        mesh=_mesh(),
        compiler_params=_sc_params(),
        scratch_types=[
            pltpu.VMEM_SHARED((N_PAD, 16), jnp.float32),
            pltpu.VMEM((NB_W, 1, LK), jnp.int32),
            pltpu.VMEM((LK, 16), jnp.float32),
        ],
    )
    def k(cols_hbm, ones_hbm, zeros_hbm, out_hbm, acc, cols_v, ones_v):
        c = lax.axis_index("c")
        s = lax.axis_index("s")
        w = c * NS + s
        pltpu.sync_copy(cols_hbm.at[w], cols_v)
        pltpu.sync_copy(ones_hbm, ones_v)
        pltpu.sync_copy(zeros_hbm, acc.at[pl.ds(s * ROWS_T, ROWS_T)])
        plsc.subcore_barrier()

        def body(nb, carry):
            pltpu.sync_copy(ones_v, acc.at[cols_v.at[nb, 0]], add=True)
            return carry

        lax.fori_loop(0, NB_W, body, 0)
        plsc.subcore_barrier()
        pltpu.sync_copy(acc.at[pl.ds(s * ROWS_T, ROWS_T)],
                        out_hbm.at[pl.ds(c * N_PAD + s * ROWS_T, ROWS_T)])

    return k(cols2, ones16, zeros16)


def _sc_gcn(hd, rows2, cols2, zeros128):
    @functools.partial(
        pl.kernel,
        out_type=jax.ShapeDtypeStruct((NC * N_PAD, H), jnp.float32),
        mesh=_mesh(),
        compiler_params=_sc_params(),
        scratch_types=[
            pltpu.VMEM_SHARED((N_PAD, H), jnp.float32),
            pltpu.VMEM((3, 1, LK), jnp.int32),
            pltpu.VMEM((3, 1, LK), jnp.int32),
            pltpu.VMEM((2 * LK, H), jnp.float32),
            pltpu.SemaphoreType.DMA,
            pltpu.SemaphoreType.DMA,
            pltpu.SemaphoreType.DMA,
            pltpu.SemaphoreType.DMA,
        ],
    )
    def k(hd_hbm, rows_hbm, cols_hbm, zeros_hbm, out_hbm,
          acc, rows_v, cols_v, buf, sem_g, sem_s, sem_ri, sem_ci):
        c = lax.axis_index("c")
        s = lax.axis_index("s")
        w = c * NS + s

        def idx_fetch(nb, ib):
            pltpu.async_copy(rows_hbm.at[w, nb], rows_v.at[ib], sem_ri)
            pltpu.async_copy(cols_hbm.at[w, nb], cols_v.at[ib], sem_ci)

        def idx_wait(nb, ib):
            pltpu.make_async_copy(rows_hbm.at[w, nb], rows_v.at[ib],
                                  sem_ri).wait()
            pltpu.make_async_copy(cols_hbm.at[w, nb], cols_v.at[ib],
                                  sem_ci).wait()

        def gather(ib, b):
            return pltpu.async_copy(hd_hbm.at[rows_v.at[ib, 0]],
                                    buf.at[pl.ds(b * LK, LK)], sem_g)

        def gather_wait(ib, b):
            pltpu.make_async_copy(hd_hbm.at[rows_v.at[ib, 0]],
                                  buf.at[pl.ds(b * LK, LK)], sem_g).wait()

        def scatter(ib, b):
            return pltpu.async_copy(buf.at[pl.ds(b * LK, LK)],
                                    acc.at[cols_v.at[ib, 0]], sem_s, add=True)

        def scatter_wait(ib, b):
            pltpu.make_async_copy(buf.at[pl.ds(b * LK, LK)],
                                  acc.at[cols_v.at[ib, 0]], sem_s).wait()

        idx_fetch(0, 0)
        pltpu.sync_copy(zeros_hbm, acc.at[pl.ds(s * ROWS_T, ROWS_T)])
        plsc.subcore_barrier()
        idx_wait(0, 0)
        gather(0, 0)
        idx_fetch(1, 1)

        def body(nb, carry):
            b = lax.rem(nb, 2)
            ib = lax.rem(nb, 3)
            gather_wait(ib, b)
            scatter(ib, b)

            @pl.when(nb + 1 < NB_W)
            def _():
                ib1 = lax.rem(nb + 1, 3)
                idx_wait(nb + 1, ib1)

                @pl.when(nb >= 1)
                def _():
                    scatter_wait(lax.rem(nb - 1, 3), 1 - b)

                gather(ib1, 1 - b)

            @pl.when(nb + 2 < NB_W)
            def _():
                idx_fetch(nb + 2, lax.rem(nb + 2, 3))

            return carry

        lax.fori_loop(0, NB_W, body, 0)
        scatter_wait(lax.rem(NB_W - 2, 3), lax.rem(NB_W - 2, 2))
        scatter_wait(lax.rem(NB_W - 1, 3), lax.rem(NB_W - 1, 2))
        plsc.subcore_barrier()
        pltpu.sync_copy(acc.at[pl.ds(s * ROWS_T, ROWS_T)],
                        out_hbm.at[pl.ds(c * N_PAD + s * ROWS_T, ROWS_T)])

    return k(hd, rows2, cols2, zeros128)


def _sc_gat(hm, as16, ad16, m16, rows2g, cols2g, zeros128, zeros16):
    @functools.partial(
        pl.kernel,
        out_type=(jax.ShapeDtypeStruct((NC * N_PAD, H), jnp.float32),
                  jax.ShapeDtypeStruct((NC * N_PAD, 16), jnp.float32)),
        mesh=_mesh(),
        compiler_params=_sc_params(),
        scratch_types=[
            pltpu.VMEM_SHARED((N_PAD, H), jnp.float32),
            pltpu.VMEM_SHARED((N_PAD, 16), jnp.float32),
            pltpu.VMEM((3, 1, LKG), jnp.int32),
            pltpu.VMEM((3, 1, LKG), jnp.int32),
            pltpu.VMEM((2 * LKG, H), jnp.float32),
            pltpu.VMEM((2 * LKG, 16), jnp.float32),
            pltpu.VMEM((2 * LKG, 16), jnp.float32),
            pltpu.VMEM((16,), jnp.float32),
            pltpu.SemaphoreType.DMA,
            pltpu.SemaphoreType.DMA,
            pltpu.SemaphoreType.DMA,
            pltpu.SemaphoreType.DMA,
            pltpu.SemaphoreType.DMA,
            pltpu.SemaphoreType.DMA,
            pltpu.SemaphoreType.DMA,
        ],
    )
    def k(hm_hbm, as_hbm, ad_hbm, m16_hbm, rows_hbm, cols_hbm,
          z128_hbm, z16_hbm, out_hbm, den_hbm,
          acc, den, rows_v, cols_v, hbuf, asb, adb, m16v,
          sem_h, sem_a, sem_b, sem_sd, sem_sa, sem_ri, sem_ci):
        c = lax.axis_index("c")
        s = lax.axis_index("s")
        w = c * NS + s

        def idx_fetch(nb, ib):
            pltpu.async_copy(rows_hbm.at[w, nb], rows_v.at[ib], sem_ri)
            pltpu.async_copy(cols_hbm.at[w, nb], cols_v.at[ib], sem_ci)

        def idx_wait(nb, ib):
            pltpu.make_async_copy(rows_hbm.at[w, nb], rows_v.at[ib],
                                  sem_ri).wait()
            pltpu.make_async_copy(cols_hbm.at[w, nb], cols_v.at[ib],
                                  sem_ci).wait()

        def gathers(ib, b):
            pltpu.async_copy(hm_hbm.at[rows_v.at[ib, 0]],
                             hbuf.at[pl.ds(b * LKG, LKG)], sem_h)
            pltpu.async_copy(as_hbm.at[rows_v.at[ib, 0]],
                             asb.at[pl.ds(b * LKG, LKG)], sem_a)
            pltpu.async_copy(ad_hbm.at[cols_v.at[ib, 0]],
                             adb.at[pl.ds(b * LKG, LKG)], sem_b)

        def gathers_wait(ib, b):
            pltpu.make_async_copy(hm_hbm.at[rows_v.at[ib, 0]],
                                  hbuf.at[pl.ds(b * LKG, LKG)], sem_h).wait()
            pltpu.make_async_copy(as_hbm.at[rows_v.at[ib, 0]],
                                  asb.at[pl.ds(b * LKG, LKG)], sem_a).wait()
            pltpu.make_async_copy(ad_hbm.at[cols_v.at[ib, 0]],
                                  adb.at[pl.ds(b * LKG, LKG)], sem_b).wait()

        def scatters(ib, b):
            pltpu.async_copy(asb.at[pl.ds(b * LKG, LKG)],
                             den.at[cols_v.at[ib, 0]], sem_sd, add=True)
            pltpu.async_copy(hbuf.at[pl.ds(b * LKG, LKG)],
                             acc.at[cols_v.at[ib, 0]], sem_sa, add=True)

        def scatters_wait(ib, b):
            pltpu.make_async_copy(asb.at[pl.ds(b * LKG, LKG)],
                                  den.at[cols_v.at[ib, 0]], sem_sd).wait()
            pltpu.make_async_copy(hbuf.at[pl.ds(b * LKG, LKG)],
                                  acc.at[cols_v.at[ib, 0]], sem_sa).wait()

        idx_fetch(0, 0)
        pltpu.sync_copy(m16_hbm, m16v)
        pltpu.sync_copy(z128_hbm, acc.at[pl.ds(s * ROWS_T, ROWS_T)])
        pltpu.sync_copy(z16_hbm, den.at[pl.ds(s * ROWS_T, ROWS_T)])
        plsc.subcore_barrier()
        idx_wait(0, 0)
        gathers(0, 0)
        idx_fetch(1, 1)
        m = m16v[...]

        def body(nb, carry):
            b = lax.rem(nb, 2)
            ib = lax.rem(nb, 3)
            gathers_wait(ib, b)

            @pl.when(nb + 1 < NBG)
            def _():
                ib1 = lax.rem(nb + 1, 3)
                idx_wait(nb + 1, ib1)

                @pl.when(nb >= 1)
                def _():
                    scatters_wait(lax.rem(nb - 1, 3), 1 - b)

                gathers(ib1, 1 - b)

            base = b * LKG

            @functools.partial(plsc.parallel_loop, 0, LKG, unroll=4)
            def _(j):
                v = asb[base + j] + adb[base + j]
                v = jnp.maximum(v, v * 0.2)
                ev = jnp.exp(v - m)
                asb[base + j] = ev
                for hd in range(HEADS):
                    blk = hbuf[base + j, pl.ds(hd * DH, DH)]
                    hbuf[base + j, pl.ds(hd * DH, DH)] = blk * ev[hd]

            scatters(ib, b)

            @pl.when(nb + 2 < NBG)
            def _():
                idx_fetch(nb + 2, lax.rem(nb + 2, 3))

            return carry

        lax.fori_loop(0, NBG, body, 0)
        scatters_wait(lax.rem(NBG - 2, 3), lax.rem(NBG - 2, 2))
        scatters_wait(lax.rem(NBG - 1, 3), lax.rem(NBG - 1, 2))
        plsc.subcore_barrier()
        pltpu.sync_copy(acc.at[pl.ds(s * ROWS_T, ROWS_T)],
                        out_hbm.at[pl.ds(c * N_PAD + s * ROWS_T, ROWS_T)])
        pltpu.sync_copy(den.at[pl.ds(s * ROWS_T, ROWS_T)],
                        den_hbm.at[pl.ds(c * N_PAD + s * ROWS_T, ROWS_T)])

    return k(hm, as16, ad16, m16, rows2g, cols2g, zeros128, zeros16)


def _graphnorm(h, p_w, p_b, p_ms):
    mean = jnp.mean(h, axis=0, keepdims=True)
    xc = h - mean * p_ms
    var = jnp.mean(xc * xc, axis=0, keepdims=True)
    return p_w * xc * lax.rsqrt(var + EPS_GN) + p_b


def _tc1(x, dega, degb, we, be, w0):
    def body(x_r, dega_r, degb_r, we_r, be_r, w0_r,
             xemb_o, hd0_o, dinv_o):
        deg = dega_r[:, 0:1] + degb_r[:, 0:1]
        dinv = jnp.where(deg > 0, lax.rsqrt(deg), 0.0)
        xemb = jnp.dot(x_r[...], we_r[...],
                       preferred_element_type=jnp.float32) + be_r[...]
        hd0 = dinv * jnp.dot(xemb, w0_r[...],
                             preferred_element_type=jnp.float32)
        xemb_o[...] = xemb
        hd0_o[...] = hd0
        dinv_o[...] = dinv

    return pl.pallas_call(
        body,
        out_shape=(jax.ShapeDtypeStruct((N, H), jnp.float32),
                   jax.ShapeDtypeStruct((N, H), jnp.float32),
                   jax.ShapeDtypeStruct((N, 1), jnp.float32)),
    )(x, dega, degb, we, be, w0)


def _tc2(p0a, p0b, dinv, b0, n0w, n0b, n0ms, xemb, w1, att_s, att_d, msel):
    def body(p0a_r, p0b_r, dinv_r, b0_r, n0w_r, n0b_r, n0ms_r, xemb_r,
             w1_r, atts_r, attd_r, msel_r,
             x1_o, hm_o, as_o, ad_o, m16_o):
        h = dinv_r[...] * (p0a_r[...] + p0b_r[...]) + b0_r[...]
        h = _graphnorm(h, n0w_r[...], n0b_r[...], n0ms_r[...])
        x1 = jnp.maximum(h, 0.0) + xemb_r[...]
        hm = jnp.dot(x1, w1_r[...], preferred_element_type=jnp.float32)
        a_s = jnp.dot(hm * atts_r[...], msel_r[...],
                      preferred_element_type=jnp.float32)
        a_d = jnp.dot(hm * attd_r[...], msel_r[...],
                      preferred_element_type=jnp.float32)
        mm = (jnp.max(a_s, axis=0, keepdims=True)
              + jnp.max(a_d, axis=0, keepdims=True))
        mub = jnp.maximum(mm, mm * 0.2)
        zero8 = jnp.zeros((N, 8), jnp.float32)
        x1_o[...] = x1
        hm_o[...] = hm
        as_o[...] = jnp.concatenate([a_s, zero8], axis=1)
        ad_o[...] = jnp.concatenate([a_d, zero8], axis=1)
        m16_o[...] = jnp.concatenate([mub, jnp.zeros((1, 8), jnp.float32)],
                                     axis=1)

    return pl.pallas_call(
        body,
        out_shape=(jax.ShapeDtypeStruct((N, H), jnp.float32),
                   jax.ShapeDtypeStruct((N, H), jnp.float32),
                   jax.ShapeDtypeStruct((N, 16), jnp.float32),
                   jax.ShapeDtypeStruct((N, 16), jnp.float32),
                   jax.ShapeDtypeStruct((1, 16), jnp.float32)),
    )(p0a, p0b, dinv, b0, n0w, n0b, n0ms, xemb, w1, att_s, att_d, msel)


def _tc4(paa, pab, dena, denb, b1, n1w, n1b, n1ms, x1, w2, dinv, mexp):
    def body(paa_r, pab_r, dena_r, denb_r, b1_r, n1w_r, n1b_r, n1ms_r,
             x1_r, w2_r, dinv_r, mexp_r, x2_o, hd2_o):
        den = dena_r[:, 0:8] + denb_r[:, 0:8]
        scale = 1.0 / (den + 1e-16)
        scale128 = jnp.dot(scale, mexp_r[...],
                           preferred_element_type=jnp.float32)
        h = (paa_r[...] + pab_r[...]) * scale128 + b1_r[...]
        h = _graphnorm(h, n1w_r[...], n1b_r[...], n1ms_r[...])
        x2 = jnp.maximum(h, 0.0) + x1_r[...]
        hd2 = dinv_r[...] * jnp.dot(x2, w2_r[...],
                                    preferred_element_type=jnp.float32)
        x2_o[...] = x2
        hd2_o[...] = hd2

    return pl.pallas_call(
        body,
        out_shape=(jax.ShapeDtypeStruct((N, H), jnp.float32),
                   jax.ShapeDtypeStruct((N, H), jnp.float32)),
    )(paa, pab, dena, denb, b1, n1w, n1b, n1ms, x1, w2, dinv, mexp)


def _tc5(p2a, p2b, dinv, b2, n2w, n2b, n2ms, x2, x, wlist):
    def body(p2a_r, p2b_r, dinv_r, b2_r, n2w_r, n2b_r, n2ms_r, x2_r, x_r,
             wi1x_r, wi1s_r, bi1_r, wi2_r, bi2_r,
             ws1x_r, ws1h_r, bs1_r, ws2_r, bs2_r,
             wp1x_r, wp1gm_r, wp1gx_r, wp1s_r, bp1_r,
             wp2_r, bp2_r, wp3_r, bp3_r,
             wt1m_r, wt1x_r, bt1_r, wt2_r, bt2_r,
             hl_o, hp_o, plg_o, pp_o, tl_o, tp_o, hi_o, xe_o):
        h = dinv_r[...] * (p2a_r[...] + p2b_r[...]) + b2_r[...]
        h = _graphnorm(h, n2w_r[...], n2b_r[...], n2ms_r[...])
        xe = jnp.maximum(h, 0.0) + x2_r[...]
        structural = x_r[:, 0:6]

        hi1 = jnp.dot(xe, wi1x_r[...], preferred_element_type=jnp.float32)
        hi1 = hi1 + jnp.dot(structural, wi1s_r[...],
                            preferred_element_type=jnp.float32) + bi1_r[...]
        hi1 = jnp.maximum(hi1, 0.0)
        himp_l = jnp.dot(hi1, wi2_r[...],
                         preferred_element_type=jnp.float32) + bi2_r[...]
        himp = 1.0 / (1.0 + jnp.exp(-himp_l))

        hs1 = jnp.dot(xe, ws1x_r[...], preferred_element_type=jnp.float32)
        hs1 = hs1 + himp * ws1h_r[...] + bs1_r[...]
        hs1 = jnp.maximum(hs1, 0.0)
        hub_l = (jnp.dot(hs1, ws2_r[...], preferred_element_type=jnp.float32)
                 + bs2_r[...] + 2.0 * himp)
        mx = jnp.max(hub_l, axis=0, keepdims=True)
        ex = jnp.exp(hub_l - mx)
        hub_p = ex / jnp.sum(ex, axis=0, keepdims=True)

        gmean = jnp.mean(xe, axis=0, keepdims=True)
        gmax = jnp.max(xe, axis=0, keepdims=True)

        pat = jnp.dot(xe, wp1x_r[...], preferred_element_type=jnp.float32)
        gfp = (jnp.dot(gmean, wp1gm_r[...], preferred_element_type=jnp.float32)
               + jnp.dot(gmax, wp1gx_r[...], preferred_element_type=jnp.float32))
        pat = pat + gfp + jnp.dot(structural, wp1s_r[...],
                                  preferred_element_type=jnp.float32) + bp1_r[...]
        pat = jnp.maximum(pat, 0.0)
        pat = jnp.maximum(jnp.dot(pat, wp2_r[...],
                                  preferred_element_type=jnp.float32)
                          + bp2_r[...], 0.0)
        pat_l = jnp.dot(pat, wp3_r[...],
                        preferred_element_type=jnp.float32) + bp3_r[...]
        pm = jnp.max(pat_l, axis=1, keepdims=True)
        pe = jnp.exp(pat_l - pm)
        pat_p = pe / jnp.sum(pe, axis=1, keepdims=True)

        t = (jnp.dot(gmean, wt1m_r[...], preferred_element_type=jnp.float32)
             + jnp.dot(gmax, wt1x_r[...], preferred_element_type=jnp.float32)
             + bt1_r[...])
        t = jnp.maximum(t, 0.0)
        term_l = jnp.dot(t, wt2_r[...],
                         preferred_element_type=jnp.float32) + bt2_r[...]
        tm = jnp.max(term_l, axis=1, keepdims=True)
        te = jnp.exp(term_l - tm)
        term_p = te / jnp.sum(te, axis=1, keepdims=True)

        hl_o[...] = hub_l
        hp_o[...] = hub_p
        plg_o[...] = pat_l
        pp_o[...] = pat_p
        tl_o[...] = term_l
        tp_o[...] = term_p
        hi_o[...] = himp
        xe_o[...] = xe

    return pl.pallas_call(
        body,
        out_shape=(jax.ShapeDtypeStruct((N, 1), jnp.float32),
                   jax.ShapeDtypeStruct((N, 1), jnp.float32),
                   jax.ShapeDtypeStruct((N, NPAT), jnp.float32),
                   jax.ShapeDtypeStruct((N, NPAT), jnp.float32),
                   jax.ShapeDtypeStruct((1, 2), jnp.float32),
                   jax.ShapeDtypeStruct((1, 2), jnp.float32),
                   jax.ShapeDtypeStruct((N, 1), jnp.float32),
                   jax.ShapeDtypeStruct((N, H), jnp.float32)),
        compiler_params=pltpu.CompilerParams(
            vmem_limit_bytes=100 * 1024 * 1024),
    )(p2a, p2b, dinv, b2, n2w, n2b, n2ms, x2, x, *wlist)


def kernel(x, edge_index, batch, params):
    del batch
    f32 = jnp.float32

    loops = jnp.arange(N, dtype=jnp.int32)
    pad = EPAD - ETOT
    rows = jnp.concatenate([edge_index[0], loops,
                            jnp.zeros((pad,), jnp.int32)])
    cols = jnp.concatenate([edge_index[1], loops,
                            jnp.full((pad,), N, jnp.int32)])
    rows2 = rows.reshape(NW, NB_W, 1, LK)
    cols2 = cols.reshape(NW, NB_W, 1, LK)
    rows2g = rows.reshape(NW, NBG, 1, LKG)
    cols2g = cols.reshape(NW, NBG, 1, LKG)

    zeros128 = jnp.zeros((ROWS_T, H), f32)
    zeros16 = jnp.zeros((ROWS_T, 16), f32)
    ones16 = jnp.ones((LK, 16), f32)

    msel = jnp.repeat(jnp.eye(HEADS, dtype=f32), DH, axis=0)
    mexp = msel.T

    p = params
    row = lambda v: v.reshape(1, -1)

    degout = _sc_degree(cols2, ones16, zeros16)
    dega = degout[0:N]
    degb = degout[N_PAD:N_PAD + N]

    xemb, hd0, dinv = _tc1(x, dega, degb,
                           p["node_embed"]["w"], row(p["node_embed"]["b"]),
                           p["gcn0"]["w"])

    p0 = _sc_gcn(hd0, rows2, cols2, zeros128)
    p0a, p0b = p0[0:N], p0[N_PAD:N_PAD + N]

    x1, hm, as16, ad16, m16 = _tc2(
        p0a, p0b, dinv, row(p["gcn0"]["b"]),
        row(p["norm0"]["weight"]), row(p["norm0"]["bias"]),
        row(p["norm0"]["mean_scale"]), xemb,
        p["gat1"]["w"], row(p["gat1"]["att_src"].reshape(-1)),
        row(p["gat1"]["att_dst"].reshape(-1)), msel)

    pa, denp = _sc_gat(hm, as16, ad16, m16.reshape(16), rows2g, cols2g,
                       zeros128, zeros16)
    paa, pab = pa[0:N], pa[N_PAD:N_PAD + N]
    dena, denb = denp[0:N], denp[N_PAD:N_PAD + N]

    x2, hd2 = _tc4(paa, pab, dena, denb, row(p["gat1"]["b"]),
                   row(p["norm1"]["weight"]), row(p["norm1"]["bias"]),
                   row(p["norm1"]["mean_scale"]), x1, p["gcn2"]["w"],
                   dinv, mexp)

    p2 = _sc_gcn(hd2, rows2, cols2, zeros128)
    p2a, p2b = p2[0:N], p2[N_PAD:N_PAD + N]

    wi1 = p["hub_imp1"]["w"]
    ws1 = p["hub_sel1"]["w"]
    pw1 = p["pat1"]["w"]
    tw1 = p["term1"]["w"]
    wlist = (
        wi1[0:H], wi1[H:H + 6], row(p["hub_imp1"]["b"]),
        p["hub_imp2"]["w"], row(p["hub_imp2"]["b"]),
        ws1[0:H], row(ws1[H]), row(p["hub_sel1"]["b"]),
        p["hub_sel2"]["w"], row(p["hub_sel2"]["b"]),
        pw1[0:H], pw1[H:2 * H], pw1[2 * H:3 * H], pw1[3 * H:3 * H + 6],
        row(p["pat1"]["b"]),
        p["pat2"]["w"], row(p["pat2"]["b"]),
        p["pat3"]["w"], row(p["pat3"]["b"]),
        tw1[0:H], tw1[H:2 * H], row(p["term1"]["b"]),
        p["term2"]["w"], row(p["term2"]["b"]),
    )
    (hub_l, hub_p, pat_l, pat_p, term_l, term_p, himp, xe) = _tc5(
        p2a, p2b, dinv, row(p["gcn2"]["b"]),
        row(p["norm2"]["weight"]), row(p["norm2"]["bias"]),
        row(p["norm2"]["mean_scale"]), x2, x, wlist)

    hub_logits = hub_l[:, 0]
    return (hub_logits, hub_p[:, 0], pat_l, pat_p,
            jnp.zeros_like(hub_logits), term_l, term_p, himp[:, 0], xe)
